# baseline (speedup 1.0000x reference)
import jax
import jax.numpy as jnp
from jax.experimental import pallas as pl
from jax.experimental.pallas import tpu as pltpu


def _prologue_kernel(x_ref, wt_ref, r1_ref, r2_ref,
                     wh_ref, e1_ref, f1_ref, e2_ref, f2_ref):
    wh = jnp.dot(x_ref[...], wt_ref[...], preferred_element_type=jnp.float32)
    wh_ref[...] = wh
    s1 = jnp.dot(wh, r1_ref[...], preferred_element_type=jnp.float32)
    s2 = jnp.dot(wh, r2_ref[...], preferred_element_type=jnp.float32)
    e1_ref[...] = jnp.exp(s1)
    f1_ref[...] = jnp.exp(0.2 * s1)
    e2_ref[...] = jnp.exp(s2)
    f2_ref[...] = jnp.exp(0.2 * s2)


def _attn_kernel(a_ref, e1_ref, f1_ref, e2_ref, f2_ref, wh_ref, out_ref):
    a = a_ref[...]
    p = jnp.maximum(e1_ref[...] * e2_ref[...], f1_ref[...] * f2_ref[...])
    attn = jnp.where(a != 0.0, p, 1.0)
    den = jnp.sum(attn * a, axis=1, keepdims=True)
    acc = jnp.dot(attn, wh_ref[...], preferred_element_type=jnp.float32)
    x = acc / den
    out_ref[...] = 0.5 * x * (1.0 + jax.lax.erf(x * 0.7071067811865476))


@jax.jit
def kernel(X, A, W, r):
    n, d_in = X.shape
    d_out = W.shape[0]

    bm = 1024

    vec = jax.ShapeDtypeStruct((n, 1), jnp.float32)
    wh, e1, f1, e2, f2 = pl.pallas_call(
        _prologue_kernel,
        grid=(1,),
        in_specs=[
            pl.BlockSpec((n, d_in), lambda i: (0, 0)),
            pl.BlockSpec((d_in, d_out), lambda i: (0, 0)),
            pl.BlockSpec((d_out, 1), lambda i: (0, 0)),
            pl.BlockSpec((d_out, 1), lambda i: (0, 0)),
        ],
        out_specs=[
            pl.BlockSpec((n, d_out), lambda i: (0, 0)),
            pl.BlockSpec((n, 1), lambda i: (0, 0)),
            pl.BlockSpec((n, 1), lambda i: (0, 0)),
            pl.BlockSpec((n, 1), lambda i: (0, 0)),
            pl.BlockSpec((n, 1), lambda i: (0, 0)),
        ],
        out_shape=[
            jax.ShapeDtypeStruct((n, d_out), jnp.float32),
            vec, vec, vec, vec,
        ],
    )(X, W.T, r[:d_out], r[d_out:])

    e2r = e2.reshape(1, n)
    f2r = f2.reshape(1, n)

    out = pl.pallas_call(
        _attn_kernel,
        grid=(n // bm,),
        in_specs=[
            pl.BlockSpec((bm, n), lambda i: (i, 0)),
            pl.BlockSpec((bm, 1), lambda i: (i, 0)),
            pl.BlockSpec((bm, 1), lambda i: (i, 0)),
            pl.BlockSpec((1, n), lambda i: (0, 0)),
            pl.BlockSpec((1, n), lambda i: (0, 0)),
            pl.BlockSpec((n, d_out), lambda i: (0, 0)),
        ],
        out_specs=pl.BlockSpec((bm, d_out), lambda i: (i, 0)),
        out_shape=jax.ShapeDtypeStruct((n, d_out), jnp.float32),
        compiler_params=pltpu.CompilerParams(
            dimension_semantics=("arbitrary",),
        ),
    )(A, e1, f1, e2r, f2r, wh)

    return out
